# baseline (speedup 1.0000x reference)
import functools

import jax
import jax.numpy as jnp
from jax import lax
from jax.experimental import pallas as pl
from jax.experimental.pallas import tpu as pltpu
from jax.experimental.pallas import tpu_sc as plsc

NUM_FIELDS = 26
VOCAB = 100000
EMBED_DIM = 64
BATCH = 16384
N_OUT_FIELDS = NUM_FIELDS + 3


def _proj_body(f, c, s, wf, bf, wc, bc, ws, bs, o):
    o[:, 0:64] = jnp.dot(f[:], wf[:], preferred_element_type=jnp.float32) + bf[:]
    o[:, 64:128] = jnp.dot(c[:], wc[:], preferred_element_type=jnp.float32) + bc[:]
    o[:, 128:192] = jnp.dot(s[:], ws[:], preferred_element_type=jnp.float32) + bs[:]


def _proj(f, c, s, wf, bf, wc, bc, ws, bs):
    BB = 1024
    grid = (BATCH // BB,)
    return pl.pallas_call(
        _proj_body,
        grid=grid,
        in_specs=[
            pl.BlockSpec((BB, 13), lambda i: (i, 0)),
            pl.BlockSpec((BB, 768), lambda i: (i, 0)),
            pl.BlockSpec((BB, 768), lambda i: (i, 0)),
            pl.BlockSpec((13, 64), lambda i: (0, 0)),
            pl.BlockSpec((1, 64), lambda i: (0, 0)),
            pl.BlockSpec((768, 64), lambda i: (0, 0)),
            pl.BlockSpec((1, 64), lambda i: (0, 0)),
            pl.BlockSpec((768, 64), lambda i: (0, 0)),
            pl.BlockSpec((1, 64), lambda i: (0, 0)),
        ],
        out_specs=pl.BlockSpec((BB, 192), lambda i: (i, 0)),
        out_shape=jax.ShapeDtypeStruct((BATCH, 192), jnp.float32),
    )(f, c, s, wf, bf, wc, bc, ws, bs)


def _sc_assemble(tables_flat, idx_t, proj3):
    info = plsc.get_sparse_core_info()
    nc, ns = info.num_cores, info.num_subcores
    nw = nc * ns
    pw = BATCH // nw
    cb = 128
    nch = pw // cb
    mesh = plsc.VectorSubcoreMesh(core_axis_name="c", subcore_axis_name="s")

    @functools.partial(
        pl.kernel,
        mesh=mesh,
        compiler_params=pltpu.CompilerParams(use_tc_tiling_on_sc=False),
        out_type=jax.ShapeDtypeStruct((BATCH, N_OUT_FIELDS, EMBED_DIM), jnp.float32),
        scratch_types=[
            pltpu.VMEM((NUM_FIELDS, cb), jnp.int32),
            pltpu.VMEM((cb, EMBED_DIM), jnp.float32),
            pltpu.VMEM((cb, EMBED_DIM), jnp.float32),
            pltpu.VMEM((cb, 3, EMBED_DIM), jnp.float32),
            pltpu.SemaphoreType.DMA,
            pltpu.SemaphoreType.DMA,
        ],
    )
    def k(tbl, idxt, proj, out, cidx, rows0, rows1, pstage, sem0, sem1):
        wid = lax.axis_index("s") * nc + lax.axis_index("c")
        base = wid * pw

        def chunk(ci, carry):
            b0 = base + ci * cb
            pltpu.sync_copy(idxt.at[:, pl.ds(b0, cb)], cidx)
            for i in range(1, NUM_FIELDS):
                for kk in range(cb // 16):
                    sl = pl.ds(kk * 16, 16)
                    cidx[i, sl] = cidx[i, sl] + jnp.int32(i * VOCAB)
            rows = (rows0, rows1)
            sems = (sem0, sem1)
            cps = [None, None]
            cps[0] = pltpu.async_copy(tbl.at[cidx.at[0]], rows[0], sems[0])
            for i in range(NUM_FIELDS):
                if i + 1 < NUM_FIELDS:
                    cps[(i + 1) % 2] = pltpu.async_copy(
                        tbl.at[cidx.at[i + 1]], rows[(i + 1) % 2], sems[(i + 1) % 2]
                    )
                cps[i % 2].wait()
                pltpu.sync_copy(rows[i % 2], out.at[pl.ds(b0, cb), i])
            pltpu.sync_copy(proj.at[pl.ds(b0, cb)], pstage)
            pltpu.sync_copy(pstage, out.at[pl.ds(b0, cb), pl.ds(NUM_FIELDS, 3)])
            return carry

        lax.fori_loop(0, nch, chunk, jnp.int32(0))

    return k(tables_flat, idx_t, proj3)


def kernel(float_inputs, idx_inputs, comment_vecs, spotlight_vecs, tables,
           W_float, b_float, W_comment, b_comment, W_spot, b_spot):
    proj = _proj(
        float_inputs, comment_vecs, spotlight_vecs,
        W_float, b_float.reshape(1, EMBED_DIM),
        W_comment, b_comment.reshape(1, EMBED_DIM),
        W_spot, b_spot.reshape(1, EMBED_DIM),
    )
    proj3 = proj.reshape(BATCH, 3, EMBED_DIM)
    tbl = tables.reshape(NUM_FIELDS * VOCAB, EMBED_DIM)
    idx_t = idx_inputs.astype(jnp.int32).T
    out = _sc_assemble(tbl, idx_t, proj3)
    return out.reshape(BATCH, N_OUT_FIELDS * EMBED_DIM)


# speedup vs baseline: 2.2287x; 2.2287x over previous
import functools

import jax
import jax.numpy as jnp
from jax import lax
from jax.experimental import pallas as pl
from jax.experimental.pallas import tpu as pltpu
from jax.experimental.pallas import tpu_sc as plsc

NUM_FIELDS = 26
VOCAB = 100000
EMBED_DIM = 64
BATCH = 16384
OUT_ROWS = NUM_FIELDS * EMBED_DIM + 3 * EMBED_DIM
PROJ_BASE = NUM_FIELDS * EMBED_DIM


def _proj_body(f, c, s, wf, wc, ws, bf, bc, bs, o):
    dn0 = (((0,), (0,)), ((), ()))
    dn1 = (((1,), (1,)), ((), ()))
    o[0:64, :] = lax.dot_general(wf[...], f[...], dn0,
                                 preferred_element_type=jnp.float32) + bf[...]
    o[64:128, :] = lax.dot_general(wc[...], c[...], dn1,
                                   preferred_element_type=jnp.float32) + bc[...]
    o[128:192, :] = lax.dot_general(ws[...], s[...], dn1,
                                    preferred_element_type=jnp.float32) + bs[...]


def _proj_t(f_t, c, s, wf, wc_t, ws_t, bf2, bc2, bs2):
    BB = 2048
    grid = (BATCH // BB,)
    return pl.pallas_call(
        _proj_body,
        grid=grid,
        in_specs=[
            pl.BlockSpec((13, BB), lambda i: (0, i)),
            pl.BlockSpec((BB, 768), lambda i: (i, 0)),
            pl.BlockSpec((BB, 768), lambda i: (i, 0)),
            pl.BlockSpec((13, 64), lambda i: (0, 0)),
            pl.BlockSpec((64, 768), lambda i: (0, 0)),
            pl.BlockSpec((64, 768), lambda i: (0, 0)),
            pl.BlockSpec((64, 1), lambda i: (0, 0)),
            pl.BlockSpec((64, 1), lambda i: (0, 0)),
            pl.BlockSpec((64, 1), lambda i: (0, 0)),
        ],
        out_specs=pl.BlockSpec((192, BB), lambda i: (0, i)),
        out_shape=jax.ShapeDtypeStruct((192, BATCH), jnp.float32),
    )(f_t, c, s, wf, wc_t, ws_t, bf2, bc2, bs2)


def _sc_lookup(tables_dmaj, idx_t, proj_t):
    mesh = plsc.VectorSubcoreMesh(core_axis_name="c", subcore_axis_name="s")
    HB = BATCH // 2

    @functools.partial(
        pl.kernel,
        mesh=mesh,
        compiler_params=pltpu.CompilerParams(
            use_tc_tiling_on_sc=True, needs_layout_passes=False),
        out_type=jax.ShapeDtypeStruct((OUT_ROWS, BATCH), jnp.float32),
        scratch_types=[
            pltpu.VMEM((VOCAB,), jnp.float32),
            pltpu.VMEM((BATCH,), jnp.int32),
            pltpu.VMEM((HB,), jnp.float32),
        ],
    )
    def k(tbl, idxt, proj, out, row_v, idx_v, res_v):
        wid = lax.axis_index("s") * 2 + lax.axis_index("c")

        def field_body(i, carry):
            pltpu.sync_copy(idxt.at[i], idx_v)

            def dim_body(dd, carry2):
                d = wid * 2 + dd
                pltpu.sync_copy(tbl.at[i, d], row_v)

                def half_body(h, carry3):
                    base = h * HB

                    def grp(g, carry4):
                        sl = pl.ds(base + g * 16, 16)
                        iv = idx_v[sl]
                        vals = plsc.load_gather(row_v, [iv])
                        res_v[pl.ds(g * 16, 16)] = vals
                        return carry4

                    lax.fori_loop(0, HB // 16, grp, 0)
                    pltpu.sync_copy(res_v, out.at[i * EMBED_DIM + d, pl.ds(base, HB)])
                    return carry3

                lax.fori_loop(0, 2, half_body, 0)
                return carry2

            lax.fori_loop(0, 2, dim_body, 0)
            return carry

        lax.fori_loop(0, NUM_FIELDS, field_body, 0)

        def proj_body(kk, carry):
            r = wid * 6 + kk
            pltpu.sync_copy(proj.at[r], row_v.at[pl.ds(0, BATCH)])
            pltpu.sync_copy(row_v.at[pl.ds(0, BATCH)], out.at[PROJ_BASE + r])
            return carry

        lax.fori_loop(0, 6, proj_body, 0)

    return k(tables_dmaj, idx_t, proj_t)


def kernel(float_inputs, idx_inputs, comment_vecs, spotlight_vecs, tables,
           W_float, b_float, W_comment, b_comment, W_spot, b_spot):
    proj_t = _proj_t(
        float_inputs.T, comment_vecs, spotlight_vecs,
        W_float, W_comment.T, W_spot.T,
        b_float.reshape(EMBED_DIM, 1), b_comment.reshape(EMBED_DIM, 1),
        b_spot.reshape(EMBED_DIM, 1),
    )
    tables_dmaj = jnp.swapaxes(tables, 1, 2)
    idx_t = idx_inputs.astype(jnp.int32).T
    out_t = _sc_lookup(tables_dmaj, idx_t, proj_t)
    return out_t.T


# speedup vs baseline: 4.1199x; 1.8486x over previous
import functools

import jax
import jax.numpy as jnp
from jax import lax
from jax.experimental import pallas as pl
from jax.experimental.pallas import tpu as pltpu
from jax.experimental.pallas import tpu_sc as plsc

NUM_FIELDS = 26
VOCAB = 100000
EMBED_DIM = 64
BATCH = 16384
OUT_ROWS = NUM_FIELDS * EMBED_DIM + 3 * EMBED_DIM
PROJ_BASE = NUM_FIELDS * EMBED_DIM


def _proj_body(f, c, s, wf, wc, ws, bf, bc, bs, o):
    dn0 = (((0,), (0,)), ((), ()))
    dn1 = (((1,), (1,)), ((), ()))
    o[0:64, :] = lax.dot_general(wf[...], f[...], dn0,
                                 preferred_element_type=jnp.float32) + bf[...]
    o[64:128, :] = lax.dot_general(wc[...], c[...], dn1,
                                   preferred_element_type=jnp.float32) + bc[...]
    o[128:192, :] = lax.dot_general(ws[...], s[...], dn1,
                                    preferred_element_type=jnp.float32) + bs[...]


def _proj_t(f_t, c, s, wf, wc_t, ws_t, bf2, bc2, bs2):
    BB = 2048
    grid = (BATCH // BB,)
    return pl.pallas_call(
        _proj_body,
        grid=grid,
        in_specs=[
            pl.BlockSpec((13, BB), lambda i: (0, i)),
            pl.BlockSpec((BB, 768), lambda i: (i, 0)),
            pl.BlockSpec((BB, 768), lambda i: (i, 0)),
            pl.BlockSpec((13, 64), lambda i: (0, 0)),
            pl.BlockSpec((64, 768), lambda i: (0, 0)),
            pl.BlockSpec((64, 768), lambda i: (0, 0)),
            pl.BlockSpec((64, 1), lambda i: (0, 0)),
            pl.BlockSpec((64, 1), lambda i: (0, 0)),
            pl.BlockSpec((64, 1), lambda i: (0, 0)),
        ],
        out_specs=pl.BlockSpec((192, BB), lambda i: (0, i)),
        out_shape=jax.ShapeDtypeStruct((192, BATCH), jnp.float32),
    )(f_t, c, s, wf, wc_t, ws_t, bf2, bc2, bs2)


def _sc_lookup(tables_dmaj, idx_t, proj_t):
    mesh = plsc.VectorSubcoreMesh(core_axis_name="c", subcore_axis_name="s")
    HB = BATCH // 2

    @functools.partial(
        pl.kernel,
        mesh=mesh,
        compiler_params=pltpu.CompilerParams(
            use_tc_tiling_on_sc=True, needs_layout_passes=False),
        out_type=jax.ShapeDtypeStruct((OUT_ROWS, BATCH), jnp.float32),
        scratch_types=[
            pltpu.VMEM((VOCAB,), jnp.float32),
            pltpu.VMEM((BATCH,), jnp.int32),
            pltpu.VMEM((HB,), jnp.float32),
        ],
    )
    def k(tbl, idxt, proj, out, row_v, idx_v, res_v):
        wid = lax.axis_index("c") * 16 + lax.axis_index("s")

        def field_body(i, carry):
            pltpu.sync_copy(idxt.at[i], idx_v)

            def dim_body(dd, carry2):
                d = wid * 2 + dd
                pltpu.sync_copy(tbl.at[i, d], row_v)

                def half_body(h, carry3):
                    base = h * HB

                    @plsc.parallel_loop(0, HB, step=16, unroll=8)
                    def grp(g):
                        iv = idx_v[pl.ds(base + g, 16)]
                        res_v[pl.ds(g, 16)] = plsc.load_gather(row_v, [iv])

                    pltpu.sync_copy(res_v, out.at[i * EMBED_DIM + d, pl.ds(base, HB)])
                    return carry3

                lax.fori_loop(0, 2, half_body, 0)
                return carry2

            lax.fori_loop(0, 2, dim_body, 0)
            return carry

        lax.fori_loop(0, NUM_FIELDS, field_body, 0)

        def proj_body(kk, carry):
            r = wid * 6 + kk
            pltpu.sync_copy(proj.at[r], row_v.at[pl.ds(0, BATCH)])
            pltpu.sync_copy(row_v.at[pl.ds(0, BATCH)], out.at[PROJ_BASE + r])
            return carry

        lax.fori_loop(0, 6, proj_body, 0)

    return k(tables_dmaj, idx_t, proj_t)


def kernel(float_inputs, idx_inputs, comment_vecs, spotlight_vecs, tables,
           W_float, b_float, W_comment, b_comment, W_spot, b_spot):
    proj_t = _proj_t(
        float_inputs.T, comment_vecs, spotlight_vecs,
        W_float, W_comment.T, W_spot.T,
        b_float.reshape(EMBED_DIM, 1), b_comment.reshape(EMBED_DIM, 1),
        b_spot.reshape(EMBED_DIM, 1),
    )
    tables_dmaj = jnp.swapaxes(tables, 1, 2)
    idx_t = idx_inputs.astype(jnp.int32).T
    out_t = _sc_lookup(tables_dmaj, idx_t, proj_t)
    return out_t.T


# speedup vs baseline: 4.3714x; 1.0610x over previous
import functools

import jax
import jax.numpy as jnp
from jax import lax
from jax.experimental import pallas as pl
from jax.experimental.pallas import tpu as pltpu
from jax.experimental.pallas import tpu_sc as plsc

NUM_FIELDS = 26
VOCAB = 100000
EMBED_DIM = 64
BATCH = 16384
OUT_ROWS = NUM_FIELDS * EMBED_DIM + 3 * EMBED_DIM
PROJ_BASE = NUM_FIELDS * EMBED_DIM


def _proj_body(f, c, s, wf, wc, ws, bf, bc, bs, o):
    dn0 = (((0,), (0,)), ((), ()))
    dn1 = (((1,), (1,)), ((), ()))
    o[0:64, :] = lax.dot_general(wf[...], f[...], dn0,
                                 preferred_element_type=jnp.float32) + bf[...]
    o[64:128, :] = lax.dot_general(wc[...], c[...], dn1,
                                   preferred_element_type=jnp.float32) + bc[...]
    o[128:192, :] = lax.dot_general(ws[...], s[...], dn1,
                                    preferred_element_type=jnp.float32) + bs[...]


def _proj_t(f_t, c, s, wf, wc_t, ws_t, bf2, bc2, bs2):
    BB = 2048
    grid = (BATCH // BB,)
    return pl.pallas_call(
        _proj_body,
        grid=grid,
        in_specs=[
            pl.BlockSpec((13, BB), lambda i: (0, i)),
            pl.BlockSpec((BB, 768), lambda i: (i, 0)),
            pl.BlockSpec((BB, 768), lambda i: (i, 0)),
            pl.BlockSpec((13, 64), lambda i: (0, 0)),
            pl.BlockSpec((64, 768), lambda i: (0, 0)),
            pl.BlockSpec((64, 768), lambda i: (0, 0)),
            pl.BlockSpec((64, 1), lambda i: (0, 0)),
            pl.BlockSpec((64, 1), lambda i: (0, 0)),
            pl.BlockSpec((64, 1), lambda i: (0, 0)),
        ],
        out_specs=pl.BlockSpec((192, BB), lambda i: (0, i)),
        out_shape=jax.ShapeDtypeStruct((192, BATCH), jnp.float32),
    )(f_t, c, s, wf, wc_t, ws_t, bf2, bc2, bs2)


def _sc_lookup(tables_dmaj, idx_t, proj_t):
    mesh = plsc.VectorSubcoreMesh(core_axis_name="c", subcore_axis_name="s")
    QB = BATCH // 4

    @functools.partial(
        pl.kernel,
        mesh=mesh,
        compiler_params=pltpu.CompilerParams(
            use_tc_tiling_on_sc=True, needs_layout_passes=False),
        out_type=jax.ShapeDtypeStruct((OUT_ROWS, BATCH), jnp.float32),
        scratch_types=[
            pltpu.VMEM((VOCAB,), jnp.float32),
            pltpu.VMEM((BATCH,), jnp.int32),
            pltpu.VMEM((QB,), jnp.float32),
            pltpu.VMEM((QB,), jnp.float32),
            pltpu.SemaphoreType.DMA,
            pltpu.SemaphoreType.DMA,
        ],
    )
    def k(tbl, idxt, proj, out, row_v, idx_v, res0_v, res1_v, sem0, sem1):
        wid = lax.axis_index("c") * 16 + lax.axis_index("s")
        res = (res0_v, res1_v)
        sems = (sem0, sem1)

        def field_body(i, carry):
            pltpu.sync_copy(idxt.at[i], idx_v)
            pend = [None, None]
            for dd in range(2):
                d = wid * 2 + dd
                pltpu.sync_copy(tbl.at[i, d], row_v)
                for q in range(4):
                    b = q % 2
                    if pend[b] is not None:
                        pend[b].wait()

                    @plsc.parallel_loop(0, QB, step=16, unroll=8)
                    def grp(g, _q=q, _b=b):
                        iv = idx_v[pl.ds(_q * QB + g, 16)]
                        res[_b][pl.ds(g, 16)] = plsc.load_gather(row_v, [iv])

                    pend[b] = pltpu.async_copy(
                        res[b], out.at[i * EMBED_DIM + d, pl.ds(q * QB, QB)],
                        sems[b])
            pend[0].wait()
            pend[1].wait()
            return carry

        lax.fori_loop(0, NUM_FIELDS, field_body, 0)

        def proj_body(kk, carry):
            r = wid * 6 + kk
            pltpu.sync_copy(proj.at[r], row_v.at[pl.ds(0, BATCH)])
            pltpu.sync_copy(row_v.at[pl.ds(0, BATCH)], out.at[PROJ_BASE + r])
            return carry

        lax.fori_loop(0, 6, proj_body, 0)

    return k(tables_dmaj, idx_t, proj_t)


def kernel(float_inputs, idx_inputs, comment_vecs, spotlight_vecs, tables,
           W_float, b_float, W_comment, b_comment, W_spot, b_spot):
    proj_t = _proj_t(
        float_inputs.T, comment_vecs, spotlight_vecs,
        W_float, W_comment.T, W_spot.T,
        b_float.reshape(EMBED_DIM, 1), b_comment.reshape(EMBED_DIM, 1),
        b_spot.reshape(EMBED_DIM, 1),
    )
    tables_dmaj = jnp.swapaxes(tables, 1, 2)
    idx_t = idx_inputs.astype(jnp.int32).T
    out_t = _sc_lookup(tables_dmaj, idx_t, proj_t)
    return out_t.T


# speedup vs baseline: 4.5345x; 1.0373x over previous
import functools

import jax
import jax.numpy as jnp
from jax import lax
from jax.experimental import pallas as pl
from jax.experimental.pallas import tpu as pltpu
from jax.experimental.pallas import tpu_sc as plsc

NUM_FIELDS = 26
VOCAB = 100000
EMBED_DIM = 64
BATCH = 16384
OUT_ROWS = NUM_FIELDS * EMBED_DIM + 3 * EMBED_DIM
PROJ_BASE = NUM_FIELDS * EMBED_DIM


def _proj_body(f, c, s, wf, wc, ws, bf, bc, bs, o):
    dn0 = (((0,), (0,)), ((), ()))
    dn1 = (((1,), (1,)), ((), ()))
    o[0:64, :] = lax.dot_general(wf[...], f[...], dn0,
                                 preferred_element_type=jnp.float32) + bf[...]
    o[64:128, :] = lax.dot_general(wc[...], c[...], dn1,
                                   preferred_element_type=jnp.float32) + bc[...]
    o[128:192, :] = lax.dot_general(ws[...], s[...], dn1,
                                    preferred_element_type=jnp.float32) + bs[...]


def _proj_t(f_t, c, s, wf, wc_t, ws_t, bf2, bc2, bs2):
    BB = 2048
    grid = (BATCH // BB,)
    return pl.pallas_call(
        _proj_body,
        grid=grid,
        in_specs=[
            pl.BlockSpec((13, BB), lambda i: (0, i)),
            pl.BlockSpec((BB, 768), lambda i: (i, 0)),
            pl.BlockSpec((BB, 768), lambda i: (i, 0)),
            pl.BlockSpec((13, 64), lambda i: (0, 0)),
            pl.BlockSpec((64, 768), lambda i: (0, 0)),
            pl.BlockSpec((64, 768), lambda i: (0, 0)),
            pl.BlockSpec((64, 1), lambda i: (0, 0)),
            pl.BlockSpec((64, 1), lambda i: (0, 0)),
            pl.BlockSpec((64, 1), lambda i: (0, 0)),
        ],
        out_specs=pl.BlockSpec((192, BB), lambda i: (0, i)),
        out_shape=jax.ShapeDtypeStruct((192, BATCH), jnp.float32),
    )(f_t, c, s, wf, wc_t, ws_t, bf2, bc2, bs2)


def _merge_proj(proj_t, out0):
    BB = 2048

    def body(p, o_any, o):
        o[...] = p[...]

    return pl.pallas_call(
        body,
        grid=(BATCH // BB, 3),
        in_specs=[
            pl.BlockSpec((EMBED_DIM, BB), lambda i, j: (j, i)),
            pl.BlockSpec(memory_space=pl.ANY),
        ],
        out_specs=pl.BlockSpec((EMBED_DIM, BB), lambda i, j: (NUM_FIELDS + j, i)),
        out_shape=jax.ShapeDtypeStruct((OUT_ROWS, BATCH), jnp.float32),
        input_output_aliases={1: 0},
    )(proj_t, out0)


def _sc_lookup(tables_dmaj, idx_t):
    mesh = plsc.VectorSubcoreMesh(core_axis_name="c", subcore_axis_name="s")
    QB = BATCH // 4

    @functools.partial(
        pl.kernel,
        mesh=mesh,
        compiler_params=pltpu.CompilerParams(
            use_tc_tiling_on_sc=True, needs_layout_passes=False),
        out_type=jax.ShapeDtypeStruct((OUT_ROWS, BATCH), jnp.float32),
        scratch_types=[
            pltpu.VMEM((VOCAB,), jnp.float32),
            pltpu.VMEM((BATCH,), jnp.int32),
            pltpu.VMEM((QB,), jnp.float32),
            pltpu.VMEM((QB,), jnp.float32),
            pltpu.SemaphoreType.DMA,
            pltpu.SemaphoreType.DMA,
        ],
    )
    def k(tbl, idxt, out, row_v, idx_v, res0_v, res1_v, sem0, sem1):
        wid = lax.axis_index("c") * 16 + lax.axis_index("s")
        res = (res0_v, res1_v)
        sems = (sem0, sem1)

        def field_body(i, carry):
            pltpu.sync_copy(idxt.at[i], idx_v)
            pend = [None, None]
            for dd in range(2):
                d = wid * 2 + dd
                pltpu.sync_copy(tbl.at[i, d], row_v)
                for q in range(4):
                    b = q % 2
                    if pend[b] is not None:
                        pend[b].wait()

                    @plsc.parallel_loop(0, QB, step=16, unroll=8)
                    def grp(g, _q=q, _b=b):
                        iv = idx_v[pl.ds(_q * QB + g, 16)]
                        res[_b][pl.ds(g, 16)] = plsc.load_gather(row_v, [iv])

                    pend[b] = pltpu.async_copy(
                        res[b], out.at[i * EMBED_DIM + d, pl.ds(q * QB, QB)],
                        sems[b])
            pend[0].wait()
            pend[1].wait()
            return carry

        lax.fori_loop(0, NUM_FIELDS, field_body, 0)

    return k(tables_dmaj, idx_t)


def kernel(float_inputs, idx_inputs, comment_vecs, spotlight_vecs, tables,
           W_float, b_float, W_comment, b_comment, W_spot, b_spot):
    tables_dmaj = jnp.swapaxes(tables, 1, 2)
    idx_t = idx_inputs.astype(jnp.int32).T
    out0 = _sc_lookup(tables_dmaj, idx_t)
    proj_t = _proj_t(
        float_inputs.T, comment_vecs, spotlight_vecs,
        W_float, W_comment.T, W_spot.T,
        b_float.reshape(EMBED_DIM, 1), b_comment.reshape(EMBED_DIM, 1),
        b_spot.reshape(EMBED_DIM, 1),
    )
    out_t = _merge_proj(proj_t, out0)
    return out_t.T


# speedup vs baseline: 4.6522x; 1.0259x over previous
import functools

import jax
import jax.numpy as jnp
from jax import lax
from jax.experimental import pallas as pl
from jax.experimental.pallas import tpu as pltpu
from jax.experimental.pallas import tpu_sc as plsc

NUM_FIELDS = 26
VOCAB = 100000
EMBED_DIM = 64
BATCH = 16384
OUT_ROWS = NUM_FIELDS * EMBED_DIM + 3 * EMBED_DIM
PROJ_BASE = NUM_FIELDS * EMBED_DIM


def _proj_body(f, c, s, wf, wc, ws, bf, bc, bs, o):
    dn0 = (((0,), (0,)), ((), ()))
    dn1 = (((1,), (1,)), ((), ()))
    o[0:64, :] = lax.dot_general(wf[...], f[...], dn0,
                                 preferred_element_type=jnp.float32) + bf[...]
    o[64:128, :] = lax.dot_general(wc[...], c[...], dn1,
                                   preferred_element_type=jnp.float32) + bc[...]
    o[128:192, :] = lax.dot_general(ws[...], s[...], dn1,
                                    preferred_element_type=jnp.float32) + bs[...]


def _proj_t(f_t, c, s, wf, wc_t, ws_t, bf2, bc2, bs2):
    BB = 2048
    grid = (BATCH // BB,)
    return pl.pallas_call(
        _proj_body,
        grid=grid,
        in_specs=[
            pl.BlockSpec((13, BB), lambda i: (0, i)),
            pl.BlockSpec((BB, 768), lambda i: (i, 0)),
            pl.BlockSpec((BB, 768), lambda i: (i, 0)),
            pl.BlockSpec((13, 64), lambda i: (0, 0)),
            pl.BlockSpec((64, 768), lambda i: (0, 0)),
            pl.BlockSpec((64, 768), lambda i: (0, 0)),
            pl.BlockSpec((64, 1), lambda i: (0, 0)),
            pl.BlockSpec((64, 1), lambda i: (0, 0)),
            pl.BlockSpec((64, 1), lambda i: (0, 0)),
        ],
        out_specs=pl.BlockSpec((192, BB), lambda i: (0, i)),
        out_shape=jax.ShapeDtypeStruct((192, BATCH), jnp.float32),
    )(f_t, c, s, wf, wc_t, ws_t, bf2, bc2, bs2)


def _merge_proj(proj_t, out0):
    BB = 8192

    def body(p, o_any, o):
        o[...] = p[...]

    return pl.pallas_call(
        body,
        grid=(BATCH // BB, 3),
        in_specs=[
            pl.BlockSpec((EMBED_DIM, BB), lambda i, j: (j, i)),
            pl.BlockSpec(memory_space=pl.ANY),
        ],
        out_specs=pl.BlockSpec((EMBED_DIM, BB), lambda i, j: (NUM_FIELDS + j, i)),
        out_shape=jax.ShapeDtypeStruct((OUT_ROWS, BATCH), jnp.float32),
        input_output_aliases={1: 0},
    )(proj_t, out0)


def _sc_lookup(tables_dmaj, idx_t):
    mesh = plsc.VectorSubcoreMesh(core_axis_name="c", subcore_axis_name="s")
    QB = BATCH // 4

    @functools.partial(
        pl.kernel,
        mesh=mesh,
        compiler_params=pltpu.CompilerParams(
            use_tc_tiling_on_sc=True, needs_layout_passes=False),
        out_type=jax.ShapeDtypeStruct((OUT_ROWS, BATCH), jnp.float32),
        scratch_types=[
            pltpu.VMEM((VOCAB,), jnp.float32),
            pltpu.VMEM((BATCH,), jnp.int32),
            pltpu.VMEM((QB,), jnp.float32),
            pltpu.VMEM((QB,), jnp.float32),
            pltpu.VMEM_SHARED((2, BATCH), jnp.int32),
            pltpu.SemaphoreType.DMA,
            pltpu.SemaphoreType.DMA,
            pltpu.SemaphoreType.DMA,
        ],
    )
    def k(tbl, idxt, out, row_v, idx_v, res0_v, res1_v, spm_idx, sem0, sem1,
          semp):
        sid = lax.axis_index("s")
        wid = lax.axis_index("c") * 16 + sid
        res = (res0_v, res1_v)
        sems = (sem0, sem1)

        @pl.when(sid == 0)
        def _():
            pltpu.sync_copy(idxt.at[0], spm_idx.at[0])

        plsc.subcore_barrier()

        def field_body(i, carry):
            pltpu.sync_copy(spm_idx.at[i % 2], idx_v)
            pend = [None, None]
            for dd in range(2):
                d = wid * 2 + dd
                pltpu.sync_copy(tbl.at[i, d], row_v)
                for q in range(4):
                    b = q % 2
                    if pend[b] is not None:
                        pend[b].wait()

                    @plsc.parallel_loop(0, QB, step=16, unroll=8)
                    def grp(g, _q=q, _b=b):
                        iv = idx_v[pl.ds(_q * QB + g, 16)]
                        res[_b][pl.ds(g, 16)] = plsc.load_gather(row_v, [iv])

                    pend[b] = pltpu.async_copy(
                        res[b], out.at[i * EMBED_DIM + d, pl.ds(q * QB, QB)],
                        sems[b])
            pend[0].wait()
            pend[1].wait()

            ip1 = jnp.minimum(i + 1, NUM_FIELDS - 1)

            @pl.when(jnp.logical_and(sid == 0, i + 1 < NUM_FIELDS))
            def _():
                pltpu.async_copy(idxt.at[ip1], spm_idx.at[(i + 1) % 2],
                                 semp).wait()

            plsc.subcore_barrier()
            return carry

        lax.fori_loop(0, NUM_FIELDS, field_body, 0)

    return k(tables_dmaj, idx_t)


def kernel(float_inputs, idx_inputs, comment_vecs, spotlight_vecs, tables,
           W_float, b_float, W_comment, b_comment, W_spot, b_spot):
    tables_dmaj = jnp.swapaxes(tables, 1, 2)
    idx_t = idx_inputs.astype(jnp.int32).T
    out0 = _sc_lookup(tables_dmaj, idx_t)
    proj_t = _proj_t(
        float_inputs.T, comment_vecs, spotlight_vecs,
        W_float, W_comment.T, W_spot.T,
        b_float.reshape(EMBED_DIM, 1), b_comment.reshape(EMBED_DIM, 1),
        b_spot.reshape(EMBED_DIM, 1),
    )
    out_t = _merge_proj(proj_t, out0)
    return out_t.T


# speedup vs baseline: 4.6688x; 1.0036x over previous
import functools

import jax
import jax.numpy as jnp
from jax import lax
from jax.experimental import pallas as pl
from jax.experimental.pallas import tpu as pltpu
from jax.experimental.pallas import tpu_sc as plsc

NUM_FIELDS = 26
VOCAB = 100000
EMBED_DIM = 64
BATCH = 16384
OUT_ROWS = NUM_FIELDS * EMBED_DIM + 3 * EMBED_DIM
PROJ_BASE = NUM_FIELDS * EMBED_DIM


def _proj_body(f, c, s, wf, wc, ws, bf, bc, bs, o):
    dn0 = (((0,), (0,)), ((), ()))
    dn1 = (((1,), (1,)), ((), ()))
    o[0:64, :] = lax.dot_general(wf[...], f[...], dn0,
                                 preferred_element_type=jnp.float32) + bf[...]
    o[64:128, :] = lax.dot_general(wc[...], c[...], dn1,
                                   preferred_element_type=jnp.float32) + bc[...]
    o[128:192, :] = lax.dot_general(ws[...], s[...], dn1,
                                    preferred_element_type=jnp.float32) + bs[...]


def _proj_t(f_t, c, s, wf, wc_t, ws_t, bf2, bc2, bs2):
    BB = 2048
    grid = (BATCH // BB,)
    return pl.pallas_call(
        _proj_body,
        grid=grid,
        in_specs=[
            pl.BlockSpec((13, BB), lambda i: (0, i)),
            pl.BlockSpec((BB, 768), lambda i: (i, 0)),
            pl.BlockSpec((BB, 768), lambda i: (i, 0)),
            pl.BlockSpec((13, 64), lambda i: (0, 0)),
            pl.BlockSpec((64, 768), lambda i: (0, 0)),
            pl.BlockSpec((64, 768), lambda i: (0, 0)),
            pl.BlockSpec((64, 1), lambda i: (0, 0)),
            pl.BlockSpec((64, 1), lambda i: (0, 0)),
            pl.BlockSpec((64, 1), lambda i: (0, 0)),
        ],
        out_specs=pl.BlockSpec((192, BB), lambda i: (0, i)),
        out_shape=jax.ShapeDtypeStruct((192, BATCH), jnp.float32),
    )(f_t, c, s, wf, wc_t, ws_t, bf2, bc2, bs2)


def _merge_proj(proj_t, out0):
    BB = 8192

    def body(p, o_any, o):
        o[...] = p[...]

    return pl.pallas_call(
        body,
        grid=(BATCH // BB, 3),
        in_specs=[
            pl.BlockSpec((EMBED_DIM, BB), lambda i, j: (j, i)),
            pl.BlockSpec(memory_space=pl.ANY),
        ],
        out_specs=pl.BlockSpec((EMBED_DIM, BB), lambda i, j: (NUM_FIELDS + j, i)),
        out_shape=jax.ShapeDtypeStruct((OUT_ROWS, BATCH), jnp.float32),
        input_output_aliases={1: 0},
    )(proj_t, out0)


def _sc_lookup(tables_dmaj, idx_t):
    mesh = plsc.VectorSubcoreMesh(core_axis_name="c", subcore_axis_name="s")
    QB = BATCH // 4

    @functools.partial(
        pl.kernel,
        mesh=mesh,
        compiler_params=pltpu.CompilerParams(
            use_tc_tiling_on_sc=True, needs_layout_passes=False),
        out_type=jax.ShapeDtypeStruct((OUT_ROWS, BATCH), jnp.float32),
        scratch_types=[
            pltpu.VMEM((VOCAB,), jnp.float32),
            pltpu.VMEM((BATCH,), jnp.int32),
            pltpu.VMEM((QB,), jnp.float32),
            pltpu.VMEM((QB,), jnp.float32),
            pltpu.VMEM_SHARED((2, BATCH), jnp.int32),
            pltpu.SemaphoreType.DMA,
            pltpu.SemaphoreType.DMA,
            pltpu.SemaphoreType.DMA,
        ],
    )
    def k(tbl, idxt, out, row_v, idx_v, res0_v, res1_v, spm_idx, sem0, sem1,
          semp):
        sid = lax.axis_index("s")
        wid = lax.axis_index("c") * 16 + sid
        res = (res0_v, res1_v)
        sems = (sem0, sem1)

        @pl.when(sid == 0)
        def _():
            pltpu.sync_copy(idxt.at[0], spm_idx.at[0])

        plsc.subcore_barrier()

        def field_body(i, carry):
            pltpu.sync_copy(spm_idx.at[i % 2], idx_v)
            pend = [None, None]
            for dd in range(2):
                d = wid * 2 + dd
                pltpu.sync_copy(tbl.at[i, d], row_v)
                for q in range(4):
                    b = q % 2
                    if pend[b] is not None:
                        pend[b].wait()

                    @plsc.parallel_loop(0, QB, step=16, unroll=16)
                    def grp(g, _q=q, _b=b):
                        iv = idx_v[pl.ds(_q * QB + g, 16)]
                        res[_b][pl.ds(g, 16)] = plsc.load_gather(row_v, [iv])

                    pend[b] = pltpu.async_copy(
                        res[b], out.at[i * EMBED_DIM + d, pl.ds(q * QB, QB)],
                        sems[b])
            pend[0].wait()
            pend[1].wait()

            ip1 = jnp.minimum(i + 1, NUM_FIELDS - 1)

            @pl.when(jnp.logical_and(sid == 0, i + 1 < NUM_FIELDS))
            def _():
                pltpu.async_copy(idxt.at[ip1], spm_idx.at[(i + 1) % 2],
                                 semp).wait()

            plsc.subcore_barrier()
            return carry

        lax.fori_loop(0, NUM_FIELDS, field_body, 0)

    return k(tables_dmaj, idx_t)


def kernel(float_inputs, idx_inputs, comment_vecs, spotlight_vecs, tables,
           W_float, b_float, W_comment, b_comment, W_spot, b_spot):
    tables_dmaj = jnp.swapaxes(tables, 1, 2)
    idx_t = idx_inputs.astype(jnp.int32).T
    out0 = _sc_lookup(tables_dmaj, idx_t)
    proj_t = _proj_t(
        float_inputs.T, comment_vecs, spotlight_vecs,
        W_float, W_comment.T, W_spot.T,
        b_float.reshape(EMBED_DIM, 1), b_comment.reshape(EMBED_DIM, 1),
        b_spot.reshape(EMBED_DIM, 1),
    )
    out_t = _merge_proj(proj_t, out0)
    return out_t.T
